# baseline (speedup 1.0000x reference)
import jax
import jax.numpy as jnp
from jax.experimental import pallas as pl
from jax.experimental.pallas import tpu as pltpu

_XR, _XC = 262144, 256
_ZR, _ZC = 16384, 1024
_XH = 16
_ZH = 8
_NB = 4
_CX = 8192
_CZ = 1024
_NXC = _XR // _CX
_NZC = _ZR // _CZ


def _stream(src, dst, bufs, in_sems, out_sems, nchunks, crows, head):
    ins, outs = [], []

    def start_in(i):
        b = i % _NB
        lo = head if i == 0 else i * crows
        n = (i + 1) * crows - lo
        c = pltpu.make_async_copy(src.at[pl.ds(lo, n), :],
                                  bufs.at[b, pl.ds(0, n), :], in_sems.at[b])
        c.start()
        ins.append((c, lo, n))

    for i in range(min(_NB, nchunks)):
        start_in(i)
    for i in range(nchunks):
        b = i % _NB
        cin, lo, n = ins[i]
        cin.wait()
        cout = pltpu.make_async_copy(bufs.at[b, pl.ds(0, n), :],
                                     dst.at[pl.ds(lo, n), :], out_sems.at[b])
        cout.start()
        outs.append(cout)
        if i + _NB < nchunks:
            outs[i].wait()
            start_in(i + _NB)
    for i in range(max(0, nchunks - _NB), nchunks):
        outs[i].wait()


def _dma_kernel(x_hbm, y_hbm, z_hbm, w_ref, xo_hbm, zo_hbm,
                xbufs, zbufs, xs, ys, zs,
                xin_sems, xout_sems, zin_sems, zout_sems, head_sems):
    hx = pltpu.make_async_copy(x_hbm.at[pl.ds(0, _XH), :], xs, head_sems.at[0])
    hy = pltpu.make_async_copy(y_hbm, ys, head_sems.at[1])
    hz = pltpu.make_async_copy(z_hbm.at[pl.ds(0, _ZH), :], zs, head_sems.at[2])
    hx.start(); hy.start(); hz.start()
    hx.wait(); hy.wait(); hz.wait()

    xv = xs[...]
    row = jax.lax.broadcasted_iota(jnp.int32, (_XH, _XC), 0)
    y0 = jnp.broadcast_to(ys[pl.ds(0, 1), :], (_XH, _XC))
    y1 = jnp.broadcast_to(ys[pl.ds(1, 1), :], (_XH, _XC))
    xv = jnp.where(row == 10, y0, xv)
    xv = jnp.where(row == 2, y1, xv)
    xv = jnp.where(row == 1, jnp.float32(45.0), xv)
    xs[...] = xv

    zrow = jax.lax.broadcasted_iota(jnp.int32, (_ZH, _ZC), 0)
    zcol = jax.lax.broadcasted_iota(jnp.int32, (_ZH, _ZC), 1)
    add = (jnp.where((zrow == 1) & (zcol == 3), w_ref[0], 0.0)
           + jnp.where((zrow == 0) & (zcol == 2), w_ref[1], 0.0)
           + jnp.where((zrow == 0) & (zcol == 1), w_ref[2], 0.0))
    zs[...] = zs[...] + add

    ox = pltpu.make_async_copy(xs, xo_hbm.at[pl.ds(0, _XH), :], head_sems.at[0])
    oz = pltpu.make_async_copy(zs, zo_hbm.at[pl.ds(0, _ZH), :], head_sems.at[1])
    ox.start(); oz.start()

    _stream(x_hbm, xo_hbm, xbufs, xin_sems, xout_sems, _NXC, _CX, _XH)
    _stream(z_hbm, zo_hbm, zbufs, zin_sems, zout_sems, _NZC, _CZ, _ZH)

    ox.wait(); oz.wait()


def kernel(x, y, z, w):
    xo, zo = pl.pallas_call(
        _dma_kernel,
        in_specs=[
            pl.BlockSpec(memory_space=pltpu.MemorySpace.HBM),
            pl.BlockSpec(memory_space=pltpu.MemorySpace.HBM),
            pl.BlockSpec(memory_space=pltpu.MemorySpace.HBM),
            pl.BlockSpec(memory_space=pltpu.MemorySpace.SMEM),
        ],
        out_specs=[
            pl.BlockSpec(memory_space=pltpu.MemorySpace.HBM),
            pl.BlockSpec(memory_space=pltpu.MemorySpace.HBM),
        ],
        out_shape=[
            jax.ShapeDtypeStruct((_XR, _XC), jnp.float32),
            jax.ShapeDtypeStruct((_ZR, _ZC), jnp.float32),
        ],
        scratch_shapes=[
            pltpu.VMEM((_NB, _CX, _XC), jnp.float32),
            pltpu.VMEM((_NB, _CZ, _ZC), jnp.float32),
            pltpu.VMEM((_XH, _XC), jnp.float32),
            pltpu.VMEM((2, _XC), jnp.float32),
            pltpu.VMEM((_ZH, _ZC), jnp.float32),
            pltpu.SemaphoreType.DMA((_NB,)),
            pltpu.SemaphoreType.DMA((_NB,)),
            pltpu.SemaphoreType.DMA((_NB,)),
            pltpu.SemaphoreType.DMA((_NB,)),
            pltpu.SemaphoreType.DMA((3,)),
        ],
    )(x, y, z, w)
    return (xo, zo)
